# baseline (speedup 1.0000x reference)
import functools

import jax
import jax.numpy as jnp
from jax import lax
from jax.experimental import pallas as pl
from jax.experimental.pallas import tpu as pltpu
from jax.experimental.pallas import tpu_sc as plsc

B, T, V, D, H = 1024, 200, 100000, 128, 64
KCH = 4
TC = T // KCH
NB = 8


def _sc_gather_chunk(seq_k, table):
    info = plsc.get_sparse_core_info()
    nw = info.num_cores * info.num_subcores
    bw = B // nw
    mesh = plsc.VectorSubcoreMesh(core_axis_name="c", subcore_axis_name="s")

    @functools.partial(
        pl.kernel,
        mesh=mesh,
        out_type=jax.ShapeDtypeStruct((B, TC, D), jnp.float32),
        scratch_types=[
            pltpu.VMEM((bw, TC), jnp.int32),
            pltpu.VMEM((TC, D), jnp.float32),
            pltpu.SemaphoreType.DMA,
        ],
    )
    def gather_kernel(seq_hbm, table_hbm, out_hbm, idx_v, rows_v, sem):
        wid = lax.axis_index("s") * info.num_cores + lax.axis_index("c")
        b0 = wid * bw
        pltpu.sync_copy(seq_hbm.at[pl.ds(b0, bw)], idx_v)

        def body(j, carry):
            pltpu.async_copy(table_hbm.at[idx_v.at[j]], rows_v, sem).wait()
            pltpu.sync_copy(rows_v, out_hbm.at[b0 + j])
            return carry

        lax.fori_loop(0, bw, body, 0)

    return gather_kernel(seq_k, table)


def _lstm_body(emb_ref, h0_ref, c0_ref, w_ref, u_ref, b_ref, obuf_ref,
               out_ref, ht_ref, ct_ref, h_s, c_s):
    del obuf_ref
    t = pl.program_id(0)

    @pl.when(t == 0)
    def _():
        h_s[...] = h0_ref[...]
        c_s[...] = c0_ref[...]

    x = emb_ref[:, 0, 0, :]
    h = h_s[...]
    z = (jnp.dot(x, w_ref[...], preferred_element_type=jnp.float32)
         + jnp.dot(h, u_ref[...], preferred_element_type=jnp.float32)
         + b_ref[...])
    i = jax.nn.sigmoid(z[:, :H])
    f = jax.nn.sigmoid(z[:, H:2 * H])
    g = jnp.tanh(z[:, 2 * H:3 * H])
    o = jax.nn.sigmoid(z[:, 3 * H:])
    c = f * c_s[...] + i * g
    hn = o * jnp.tanh(c)
    h_s[...] = hn
    c_s[...] = c
    out_ref[:, 0, 0, :] = hn

    @pl.when(t == TC - 1)
    def _():
        ht_ref[...] = hn
        ct_ref[...] = c


def _tc_lstm_chunk(k, emb_k4, h, c, w, u, b2d, out_buf):
    return pl.pallas_call(
        _lstm_body,
        grid=(TC,),
        in_specs=[
            pl.BlockSpec((B, 1, 1, D), lambda t: (0, t, 0, 0)),
            pl.BlockSpec((B, H), lambda t: (0, 0)),
            pl.BlockSpec((B, H), lambda t: (0, 0)),
            pl.BlockSpec((D, 4 * H), lambda t: (0, 0)),
            pl.BlockSpec((H, 4 * H), lambda t: (0, 0)),
            pl.BlockSpec((1, 4 * H), lambda t: (0, 0)),
            pl.BlockSpec(memory_space=pl.ANY),
        ],
        out_specs=[
            pl.BlockSpec((B, 1, 1, H), lambda t: (0, k * TC + t, 0, 0)),
            pl.BlockSpec((B, H), lambda t: (0, 0)),
            pl.BlockSpec((B, H), lambda t: (0, 0)),
        ],
        out_shape=[
            jax.ShapeDtypeStruct((B, T, 1, H), jnp.float32),
            jax.ShapeDtypeStruct((B, H), jnp.float32),
            jax.ShapeDtypeStruct((B, H), jnp.float32),
        ],
        scratch_shapes=[
            pltpu.VMEM((B, H), jnp.float32),
            pltpu.VMEM((B, H), jnp.float32),
        ],
        input_output_aliases={6: 0},
    )(emb_k4, h, c, w, u, b2d, out_buf)


def kernel(sequence, states_1, states_2, table, W, U, b):
    b2d = b.reshape(1, 4 * H)
    out_buf = jnp.zeros((B, T, 1, H), jnp.float32)
    h, c = states_1, states_2
    for k in range(KCH):
        seq_k = lax.slice_in_dim(sequence, k * TC, (k + 1) * TC, axis=1)
        emb_k = _sc_gather_chunk(seq_k, table).reshape(B, TC, 1, D)
        out_buf, h, c = _tc_lstm_chunk(k, emb_k, h, c, W, U, b2d, out_buf)
    return out_buf.reshape(B, T, H), h, c


# speedup vs baseline: 1.2943x; 1.2943x over previous
import functools

import jax
import jax.numpy as jnp
from jax import lax
from jax.experimental import pallas as pl
from jax.experimental.pallas import tpu as pltpu
from jax.experimental.pallas import tpu_sc as plsc

B, T, V, D, H = 1024, 200, 100000, 128, 64
KCH = 5
TC = T // KCH
CH = 80


def _sc_gather_chunk(idx2d, table):
    info = plsc.get_sparse_core_info()
    nw = info.num_cores * info.num_subcores
    per_w = (TC * B) // nw
    n_ch = per_w // CH
    mesh = plsc.VectorSubcoreMesh(core_axis_name="c", subcore_axis_name="s")

    @functools.partial(
        pl.kernel,
        mesh=mesh,
        out_type=jax.ShapeDtypeStruct((TC * B, D), jnp.float32),
        scratch_types=[
            pltpu.VMEM((n_ch, CH), jnp.int32),
            pltpu.VMEM((CH, D), jnp.float32),
            pltpu.VMEM((CH, D), jnp.float32),
            pltpu.SemaphoreType.DMA,
            pltpu.SemaphoreType.DMA,
        ],
    )
    def gather_kernel(idx_hbm, table_hbm, out_hbm, idx_v, buf_a, buf_b,
                      sem_a, sem_b):
        wid = lax.axis_index("s") * info.num_cores + lax.axis_index("c")
        r0 = wid * per_w
        pltpu.sync_copy(idx_hbm.at[pl.ds(wid * n_ch, n_ch)], idx_v)

        def gather_to(j, buf, sem):
            pltpu.make_async_copy(table_hbm.at[idx_v.at[j]], buf, sem).start()

        def wait_for(j, buf, sem):
            pltpu.make_async_copy(table_hbm.at[idx_v.at[j]], buf, sem).wait()

        gather_to(0, buf_a, sem_a)

        def body(jj, carry):
            j = jj * 2
            wait_for(j, buf_a, sem_a)
            gather_to(j + 1, buf_b, sem_b)
            pltpu.sync_copy(buf_a, out_hbm.at[pl.ds(r0 + j * CH, CH)])
            wait_for(j + 1, buf_b, sem_b)

            @pl.when(j + 2 < n_ch)
            def _():
                gather_to(j + 2, buf_a, sem_a)

            pltpu.sync_copy(buf_b, out_hbm.at[pl.ds(r0 + (j + 1) * CH, CH)])
            return carry

        lax.fori_loop(0, n_ch // 2, body, 0)

    return gather_kernel(idx2d, table)


def _lstm_body(emb_ref, h0_ref, c0_ref, w_ref, u_ref, b_ref, obuf_ref,
               out_ref, ht_ref, ct_ref, h_s, c_s):
    del obuf_ref
    t = pl.program_id(0)

    @pl.when(t == 0)
    def _():
        h_s[...] = h0_ref[...]
        c_s[...] = c0_ref[...]

    x = emb_ref[0]
    h = h_s[...]
    z = (jnp.dot(x, w_ref[...], preferred_element_type=jnp.float32)
         + jnp.dot(h, u_ref[...], preferred_element_type=jnp.float32)
         + b_ref[...])
    i = jax.nn.sigmoid(z[:, :H])
    f = jax.nn.sigmoid(z[:, H:2 * H])
    g = jnp.tanh(z[:, 2 * H:3 * H])
    o = jax.nn.sigmoid(z[:, 3 * H:])
    c = f * c_s[...] + i * g
    hn = o * jnp.tanh(c)
    h_s[...] = hn
    c_s[...] = c
    out_ref[:, t, :] = hn

    @pl.when(t == TC - 1)
    def _():
        ht_ref[...] = hn
        ct_ref[...] = c


def _tc_lstm_chunk(k, emb_k, h, c, w, u, b2d, out_buf):
    return pl.pallas_call(
        _lstm_body,
        grid=(TC,),
        in_specs=[
            pl.BlockSpec((1, B, D), lambda t: (t, 0, 0)),
            pl.BlockSpec((B, H), lambda t: (0, 0)),
            pl.BlockSpec((B, H), lambda t: (0, 0)),
            pl.BlockSpec((D, 4 * H), lambda t: (0, 0)),
            pl.BlockSpec((H, 4 * H), lambda t: (0, 0)),
            pl.BlockSpec((1, 4 * H), lambda t: (0, 0)),
            pl.BlockSpec(memory_space=pl.ANY),
        ],
        out_specs=[
            pl.BlockSpec((B, TC, H), lambda t: (0, k, 0)),
            pl.BlockSpec((B, H), lambda t: (0, 0)),
            pl.BlockSpec((B, H), lambda t: (0, 0)),
        ],
        out_shape=[
            jax.ShapeDtypeStruct((B, T, H), jnp.float32),
            jax.ShapeDtypeStruct((B, H), jnp.float32),
            jax.ShapeDtypeStruct((B, H), jnp.float32),
        ],
        scratch_shapes=[
            pltpu.VMEM((B, H), jnp.float32),
            pltpu.VMEM((B, H), jnp.float32),
        ],
        input_output_aliases={6: 0},
    )(emb_k, h, c, w, u, b2d, out_buf)


def kernel(sequence, states_1, states_2, table, W, U, b):
    b2d = b.reshape(1, 4 * H)
    out_buf = jnp.zeros((B, T, H), jnp.float32)
    h, c = states_1, states_2
    for k in range(KCH):
        idx2d = jnp.transpose(
            lax.slice_in_dim(sequence, k * TC, (k + 1) * TC, axis=1)
        ).reshape(TC * B // CH, CH)
        emb_k = _sc_gather_chunk(idx2d, table).reshape(TC, B, D)
        out_buf, h, c = _tc_lstm_chunk(k, emb_k, h, c, W, U, b2d, out_buf)
    return out_buf, h, c


# speedup vs baseline: 1.6031x; 1.2386x over previous
import functools

import jax
import jax.numpy as jnp
from jax import lax
from jax.experimental import pallas as pl
from jax.experimental.pallas import tpu as pltpu
from jax.experimental.pallas import tpu_sc as plsc

B, T, V, D, H = 1024, 200, 100000, 128, 64
KCH = 5
TC = T // KCH
TSUB = 8
NGC = TC // TSUB
CH = 80


def _sc_gather_chunk(idx2d, table):
    info = plsc.get_sparse_core_info()
    nw = info.num_cores * info.num_subcores
    per_w = (TC * B) // nw
    n_ch = per_w // CH
    mesh = plsc.VectorSubcoreMesh(core_axis_name="c", subcore_axis_name="s")

    @functools.partial(
        pl.kernel,
        mesh=mesh,
        out_type=jax.ShapeDtypeStruct((TC * B, D), jnp.float32),
        scratch_types=[
            pltpu.VMEM((n_ch, CH), jnp.int32),
            pltpu.VMEM((CH, D), jnp.float32),
            pltpu.VMEM((CH, D), jnp.float32),
            pltpu.SemaphoreType.DMA,
            pltpu.SemaphoreType.DMA,
        ],
    )
    def gather_kernel(idx_hbm, table_hbm, out_hbm, idx_v, buf_a, buf_b,
                      sem_a, sem_b):
        wid = lax.axis_index("s") * info.num_cores + lax.axis_index("c")
        r0 = wid * per_w
        pltpu.sync_copy(idx_hbm.at[pl.ds(wid * n_ch, n_ch)], idx_v)

        def gather_to(j, buf, sem):
            pltpu.make_async_copy(table_hbm.at[idx_v.at[j]], buf, sem).start()

        def wait_for(j, buf, sem):
            pltpu.make_async_copy(table_hbm.at[idx_v.at[j]], buf, sem).wait()

        gather_to(0, buf_a, sem_a)

        def body(jj, carry):
            j = jj * 2
            wait_for(j, buf_a, sem_a)
            gather_to(j + 1, buf_b, sem_b)
            pltpu.sync_copy(buf_a, out_hbm.at[pl.ds(r0 + j * CH, CH)])
            wait_for(j + 1, buf_b, sem_b)

            @pl.when(j + 2 < n_ch)
            def _():
                gather_to(j + 2, buf_a, sem_a)

            pltpu.sync_copy(buf_b, out_hbm.at[pl.ds(r0 + (j + 1) * CH, CH)])
            return carry

        lax.fori_loop(0, n_ch // 2, body, 0)

    return gather_kernel(idx2d, table)


def _lstm_body(emb_ref, h0_ref, c0_ref, w_ref, u_ref, b_ref, obuf_ref,
               out_ref, ht_ref, ct_ref, h_s, c_s):
    del obuf_ref
    g = pl.program_id(0)

    @pl.when(g == 0)
    def _():
        h_s[...] = h0_ref[...]
        c_s[...] = c0_ref[...]

    u = u_ref[...]
    xz = jnp.dot(emb_ref[...].reshape(TSUB * B, D), w_ref[...],
                 preferred_element_type=jnp.float32).reshape(TSUB, B, 4 * H)
    xz = xz + b_ref[...]
    h = h_s[...]
    c = c_s[...]
    for j in range(TSUB):
        z = xz[j] + jnp.dot(h, u, preferred_element_type=jnp.float32)
        i = jax.nn.sigmoid(z[:, :H])
        f = jax.nn.sigmoid(z[:, H:2 * H])
        gg = jnp.tanh(z[:, 2 * H:3 * H])
        o = jax.nn.sigmoid(z[:, 3 * H:])
        c = f * c + i * gg
        h = o * jnp.tanh(c)
        out_ref[:, j, :] = h
    h_s[...] = h
    c_s[...] = c

    @pl.when(g == NGC - 1)
    def _():
        ht_ref[...] = h
        ct_ref[...] = c


def _tc_lstm_chunk(k, emb_k, h, c, w, u, b2d, out_buf):
    in_specs = [
        pl.BlockSpec((TSUB, B, D), lambda g: (g, 0, 0)),
        pl.BlockSpec((B, H), lambda g: (0, 0)),
        pl.BlockSpec((B, H), lambda g: (0, 0)),
        pl.BlockSpec((D, 4 * H), lambda g: (0, 0)),
        pl.BlockSpec((H, 4 * H), lambda g: (0, 0)),
        pl.BlockSpec((1, 4 * H), lambda g: (0, 0)),
    ]
    operands = [emb_k, h, c, w, u, b2d]
    aliases = {}
    body = _lstm_body
    if out_buf is None:
        def body(emb, h0, c0, w_, u_, bb, out, ht, ct, hs, cs):
            return _lstm_body(emb, h0, c0, w_, u_, bb, None, out, ht, ct,
                              hs, cs)
    else:
        in_specs.append(pl.BlockSpec(memory_space=pl.ANY))
        operands.append(out_buf)
        aliases = {6: 0}
    return pl.pallas_call(
        body,
        grid=(NGC,),
        in_specs=in_specs,
        out_specs=[
            pl.BlockSpec((B, TSUB, H), lambda g, _k=k: (0, _k * NGC + g, 0)),
            pl.BlockSpec((B, H), lambda g: (0, 0)),
            pl.BlockSpec((B, H), lambda g: (0, 0)),
        ],
        out_shape=[
            jax.ShapeDtypeStruct((B, T, H), jnp.float32),
            jax.ShapeDtypeStruct((B, H), jnp.float32),
            jax.ShapeDtypeStruct((B, H), jnp.float32),
        ],
        scratch_shapes=[
            pltpu.VMEM((B, H), jnp.float32),
            pltpu.VMEM((B, H), jnp.float32),
        ],
        input_output_aliases=aliases,
    )(*operands)


def kernel(sequence, states_1, states_2, table, W, U, b):
    b2d = b.reshape(1, 4 * H)
    out_buf = None
    h, c = states_1, states_2
    for k in range(KCH):
        idx2d = jnp.transpose(
            lax.slice_in_dim(sequence, k * TC, (k + 1) * TC, axis=1)
        ).reshape(TC * B // CH, CH)
        emb_k = _sc_gather_chunk(idx2d, table).reshape(TC, B, D)
        out_buf, h, c = _tc_lstm_chunk(k, emb_k, h, c, W, U, b2d, out_buf)
    return out_buf, h, c


# speedup vs baseline: 2.9477x; 1.8387x over previous
import functools

import jax
import jax.numpy as jnp
from jax import lax
from jax.experimental import pallas as pl
from jax.experimental.pallas import tpu as pltpu
from jax.experimental.pallas import tpu_sc as plsc

B, T, V, D, H = 1024, 200, 100000, 128, 64
KCH = 5
TC = T // KCH
TSUB = 8
NGC = TC // TSUB
CH = 80


def _sc_gather_chunk(idx2d, table):
    info = plsc.get_sparse_core_info()
    nw = info.num_cores * info.num_subcores
    per_w = (TC * B) // nw
    n_ch = per_w // CH
    mesh = plsc.VectorSubcoreMesh(core_axis_name="c", subcore_axis_name="s")

    @functools.partial(
        pl.kernel,
        mesh=mesh,
        out_type=jax.ShapeDtypeStruct((TC * B, D), jnp.float32),
        scratch_types=[
            pltpu.VMEM((n_ch, CH), jnp.int32),
            pltpu.VMEM((CH, D), jnp.float32),
            pltpu.VMEM((CH, D), jnp.float32),
            pltpu.SemaphoreType.DMA,
            pltpu.SemaphoreType.DMA,
        ],
    )
    def gather_kernel(idx_hbm, table_hbm, out_hbm, idx_v, buf_a, buf_b,
                      sem_a, sem_b):
        wid = lax.axis_index("s") * info.num_cores + lax.axis_index("c")
        r0 = wid * per_w
        pltpu.sync_copy(idx_hbm.at[pl.ds(wid * n_ch, n_ch)], idx_v)

        def gather_to(j, buf, sem):
            pltpu.make_async_copy(table_hbm.at[idx_v.at[j]], buf, sem).start()

        def wait_for(j, buf, sem):
            pltpu.make_async_copy(table_hbm.at[idx_v.at[j]], buf, sem).wait()

        gather_to(0, buf_a, sem_a)

        def body(jj, carry):
            j = jj * 2
            wait_for(j, buf_a, sem_a)
            gather_to(j + 1, buf_b, sem_b)
            pltpu.sync_copy(buf_a, out_hbm.at[pl.ds(r0 + j * CH, CH)])
            wait_for(j + 1, buf_b, sem_b)

            @pl.when(j + 2 < n_ch)
            def _():
                gather_to(j + 2, buf_a, sem_a)

            pltpu.sync_copy(buf_b, out_hbm.at[pl.ds(r0 + (j + 1) * CH, CH)])
            return carry

        lax.fori_loop(0, n_ch // 2, body, 0)

    return gather_kernel(idx2d, table)


def _lstm_body(emb_ref, h0_ref, c0_ref, w_ref, u_ref, b_ref, obuf_ref,
               out_ref, ht_ref, ct_ref, h_s, c_s):
    del obuf_ref
    g = pl.program_id(0)

    @pl.when(g == 0)
    def _():
        h_s[...] = h0_ref[...]
        c_s[...] = c0_ref[...]

    u = u_ref[...]
    w = w_ref[...]
    bb = b_ref[...]
    h = h_s[...]
    c = c_s[...]
    h_prev = h
    for j in range(TSUB):
        z = (jnp.dot(emb_ref[j], w, preferred_element_type=jnp.float32)
             + jnp.dot(h, u, preferred_element_type=jnp.float32) + bb)
        i = jax.nn.sigmoid(z[:, :H])
        f = jax.nn.sigmoid(z[:, H:2 * H])
        gg = jnp.tanh(z[:, 2 * H:3 * H])
        o = jax.nn.sigmoid(z[:, 3 * H:])
        c = f * c + i * gg
        h = o * jnp.tanh(c)
        if j % 2 == 0:
            h_prev = h
        else:
            out_ref[:, (j // 2) * 2 * H:(j // 2 + 1) * 2 * H] = (
                jnp.concatenate([h_prev, h], axis=1))
    h_s[...] = h
    c_s[...] = c

    @pl.when(g == NGC - 1)
    def _():
        ht_ref[...] = h
        ct_ref[...] = c


def _tc_lstm_chunk(k, emb_k, h, c, w, u, b2d, out_buf):
    in_specs = [
        pl.BlockSpec((TSUB, B, D), lambda g: (g, 0, 0)),
        pl.BlockSpec((B, H), lambda g: (0, 0)),
        pl.BlockSpec((B, H), lambda g: (0, 0)),
        pl.BlockSpec((D, 4 * H), lambda g: (0, 0)),
        pl.BlockSpec((H, 4 * H), lambda g: (0, 0)),
        pl.BlockSpec((1, 4 * H), lambda g: (0, 0)),
    ]
    operands = [emb_k, h, c, w, u, b2d]
    aliases = {}
    body = _lstm_body
    if out_buf is None:
        def body(emb, h0, c0, w_, u_, bb, out, ht, ct, hs, cs):
            return _lstm_body(emb, h0, c0, w_, u_, bb, None, out, ht, ct,
                              hs, cs)
    else:
        in_specs.append(pl.BlockSpec(memory_space=pl.ANY))
        operands.append(out_buf)
        aliases = {6: 0}
    return pl.pallas_call(
        body,
        grid=(NGC,),
        in_specs=in_specs,
        out_specs=[
            pl.BlockSpec((B, TSUB * H), lambda g, _k=k: (0, _k * NGC + g)),
            pl.BlockSpec((B, H), lambda g: (0, 0)),
            pl.BlockSpec((B, H), lambda g: (0, 0)),
        ],
        out_shape=[
            jax.ShapeDtypeStruct((B, T * H), jnp.float32),
            jax.ShapeDtypeStruct((B, H), jnp.float32),
            jax.ShapeDtypeStruct((B, H), jnp.float32),
        ],
        scratch_shapes=[
            pltpu.VMEM((B, H), jnp.float32),
            pltpu.VMEM((B, H), jnp.float32),
        ],
        input_output_aliases=aliases,
    )(*operands)


def kernel(sequence, states_1, states_2, table, W, U, b):
    b2d = b.reshape(1, 4 * H)
    out_buf = None
    h, c = states_1, states_2
    for k in range(KCH):
        idx2d = jnp.transpose(
            lax.slice_in_dim(sequence, k * TC, (k + 1) * TC, axis=1)
        ).reshape(TC * B // CH, CH)
        emb_k = _sc_gather_chunk(idx2d, table).reshape(TC, B, D)
        out_buf, h, c = _tc_lstm_chunk(k, emb_k, h, c, W, U, b2d, out_buf)
    return out_buf.reshape(B, T, H), h, c


# speedup vs baseline: 3.7485x; 1.2717x over previous
import functools

import jax
import jax.numpy as jnp
from jax import lax
from jax.experimental import pallas as pl
from jax.experimental.pallas import tpu as pltpu
from jax.experimental.pallas import tpu_sc as plsc

B, T, V, D, H = 1024, 200, 100000, 128, 64
KCH = 5
TC = T // KCH
TSUB = 8
NGC = TC // TSUB
CH = 80


def _sc_gather_chunk(idx2d, table):
    info = plsc.get_sparse_core_info()
    nw = info.num_cores * info.num_subcores
    per_w = (TC * B) // nw
    n_ch = per_w // CH
    mesh = plsc.VectorSubcoreMesh(core_axis_name="c", subcore_axis_name="s")

    @functools.partial(
        pl.kernel,
        mesh=mesh,
        out_type=jax.ShapeDtypeStruct((TC * B, D), jnp.float32),
        scratch_types=[
            pltpu.VMEM((n_ch, CH), jnp.int32),
            pltpu.VMEM((CH, D), jnp.float32),
            pltpu.VMEM((CH, D), jnp.float32),
            pltpu.SemaphoreType.DMA,
            pltpu.SemaphoreType.DMA,
        ],
    )
    def gather_kernel(idx_hbm, table_hbm, out_hbm, idx_v, buf_a, buf_b,
                      sem_a, sem_b):
        wid = lax.axis_index("s") * info.num_cores + lax.axis_index("c")
        r0 = wid * per_w
        pltpu.sync_copy(idx_hbm.at[pl.ds(wid * n_ch, n_ch)], idx_v)

        def gather_to(j, buf, sem):
            pltpu.make_async_copy(table_hbm.at[idx_v.at[j]], buf, sem).start()

        def wait_for(j, buf, sem):
            pltpu.make_async_copy(table_hbm.at[idx_v.at[j]], buf, sem).wait()

        gather_to(0, buf_a, sem_a)

        def body(jj, carry):
            j = jj * 2
            wait_for(j, buf_a, sem_a)
            gather_to(j + 1, buf_b, sem_b)
            pltpu.sync_copy(buf_a, out_hbm.at[pl.ds(r0 + j * CH, CH)])
            wait_for(j + 1, buf_b, sem_b)

            @pl.when(j + 2 < n_ch)
            def _():
                gather_to(j + 2, buf_a, sem_a)

            pltpu.sync_copy(buf_b, out_hbm.at[pl.ds(r0 + (j + 1) * CH, CH)])
            return carry

        lax.fori_loop(0, n_ch // 2, body, 0)

    return gather_kernel(idx2d, table)


def _lstm_body(emb_ref, h0_ref, c0_ref, wt_ref, ut_ref, b_ref, obuf_ref,
               out_ref, ht_ref, ct_ref, h_s, c_s):
    del obuf_ref
    g = pl.program_id(0)

    @pl.when(g == 0)
    def _():
        h_s[...] = h0_ref[...]
        c_s[...] = c0_ref[...]

    ut = ut_ref[...]
    wt = wt_ref[...]
    bb = b_ref[...]
    h = h_s[...]
    c = c_s[...]
    nt = (((1,), (1,)), ((), ()))
    for j in range(TSUB):
        z = (lax.dot_general(wt, emb_ref[j], nt,
                             preferred_element_type=jnp.float32)
             + jnp.dot(ut, h, preferred_element_type=jnp.float32) + bb)
        s_if = jax.nn.sigmoid(z[:2 * H, :])
        i = s_if[:H, :]
        f = s_if[H:, :]
        gg = jnp.tanh(z[2 * H:3 * H, :])
        o = jax.nn.sigmoid(z[3 * H:, :])
        c = f * c + i * gg
        h = o * jnp.tanh(c)
        out_ref[j * H:(j + 1) * H, :] = h
    h_s[...] = h
    c_s[...] = c

    @pl.when(g == NGC - 1)
    def _():
        ht_ref[...] = h
        ct_ref[...] = c


def _tc_lstm_chunk(k, emb_k, ht, ct, wt, ut, b2d, out_buf):
    in_specs = [
        pl.BlockSpec((TSUB, B, D), lambda g: (g, 0, 0)),
        pl.BlockSpec((H, B), lambda g: (0, 0)),
        pl.BlockSpec((H, B), lambda g: (0, 0)),
        pl.BlockSpec((4 * H, D), lambda g: (0, 0)),
        pl.BlockSpec((4 * H, H), lambda g: (0, 0)),
        pl.BlockSpec((4 * H, 1), lambda g: (0, 0)),
    ]
    operands = [emb_k, ht, ct, wt, ut, b2d]
    aliases = {}
    body = _lstm_body
    if out_buf is None:
        def body(emb, h0, c0, w_, u_, bb, out, ho, co, hs, cs):
            return _lstm_body(emb, h0, c0, w_, u_, bb, None, out, ho, co,
                              hs, cs)
    else:
        in_specs.append(pl.BlockSpec(memory_space=pl.ANY))
        operands.append(out_buf)
        aliases = {6: 0}
    return pl.pallas_call(
        body,
        grid=(NGC,),
        in_specs=in_specs,
        out_specs=[
            pl.BlockSpec((TSUB * H, B), lambda g, _k=k: (_k * NGC + g, 0)),
            pl.BlockSpec((H, B), lambda g: (0, 0)),
            pl.BlockSpec((H, B), lambda g: (0, 0)),
        ],
        out_shape=[
            jax.ShapeDtypeStruct((T * H, B), jnp.float32),
            jax.ShapeDtypeStruct((H, B), jnp.float32),
            jax.ShapeDtypeStruct((H, B), jnp.float32),
        ],
        scratch_shapes=[
            pltpu.VMEM((H, B), jnp.float32),
            pltpu.VMEM((H, B), jnp.float32),
        ],
        input_output_aliases=aliases,
    )(*operands)


def kernel(sequence, states_1, states_2, table, W, U, b):
    wt = jnp.transpose(W)
    ut = jnp.transpose(U)
    bt = b.reshape(4 * H, 1)
    ht = jnp.transpose(states_1)
    ct = jnp.transpose(states_2)
    out_buf = None
    for k in range(KCH):
        idx2d = jnp.transpose(
            lax.slice_in_dim(sequence, k * TC, (k + 1) * TC, axis=1)
        ).reshape(TC * B // CH, CH)
        emb_k = _sc_gather_chunk(idx2d, table).reshape(TC, B, D)
        out_buf, ht, ct = _tc_lstm_chunk(k, emb_k, ht, ct, wt, ut, bt,
                                         out_buf)
    out = jnp.transpose(out_buf.reshape(T, H, B), (2, 0, 1))
    return out, jnp.transpose(ht), jnp.transpose(ct)


# speedup vs baseline: 4.2768x; 1.1409x over previous
import functools

import jax
import jax.numpy as jnp
from jax import lax
from jax.experimental import pallas as pl
from jax.experimental.pallas import tpu as pltpu
from jax.experimental.pallas import tpu_sc as plsc

B, T, V, D, H = 1024, 200, 100000, 128, 64
KCH = 4
TC = T // KCH
TSUB = 10
NGC = TC // TSUB
CH = 80
NBUF = 4


def _sc_gather_chunk(idx2d, table):
    info = plsc.get_sparse_core_info()
    nw = info.num_cores * info.num_subcores
    per_w = (TC * B) // nw
    n_ch = per_w // CH
    mesh = plsc.VectorSubcoreMesh(core_axis_name="c", subcore_axis_name="s")

    @functools.partial(
        pl.kernel,
        mesh=mesh,
        out_type=jax.ShapeDtypeStruct((TC * B, D), jnp.float32),
        scratch_types=[
            pltpu.VMEM((n_ch, CH), jnp.int32),
            pltpu.VMEM((NBUF, CH, D), jnp.float32),
        ] + [pltpu.SemaphoreType.DMA] * (2 * NBUF),
    )
    def gather_kernel(idx_hbm, table_hbm, out_hbm, idx_v, bufs, *sems):
        gsem, wsem = sems[:NBUF], sems[NBUF:]
        wid = lax.axis_index("s") * info.num_cores + lax.axis_index("c")
        r0 = wid * per_w
        pltpu.sync_copy(idx_hbm.at[wid], idx_v)

        def g_copy(j):
            r = j % NBUF
            return pltpu.make_async_copy(
                table_hbm.at[idx_v.at[j]], bufs.at[r], gsem[r])

        def w_copy(j):
            r = j % NBUF
            return pltpu.make_async_copy(
                bufs.at[r], out_hbm.at[pl.ds(r0 + j * CH, CH)], wsem[r])

        g_copy(0).start()
        g_copy(1).start()
        for j in range(n_ch):
            g_copy(j).wait()
            w_copy(j).start()
            if j + 2 < n_ch:
                if j >= 2:
                    w_copy(j - 2).wait()
                g_copy(j + 2).start()
        w_copy(n_ch - 4).wait()
        w_copy(n_ch - 3).wait()
        w_copy(n_ch - 2).wait()
        w_copy(n_ch - 1).wait()

    return gather_kernel(idx2d, table)


def _lstm_body(emb_ref, h0_ref, c0_ref, wt_ref, ut_ref, b_ref, obuf_ref,
               out_ref, ht_ref, ct_ref, h_s, c_s):
    del obuf_ref
    g = pl.program_id(0)

    @pl.when(g == 0)
    def _():
        h_s[...] = h0_ref[...]
        c_s[...] = c0_ref[...]

    ut = ut_ref[...]
    wt = wt_ref[...]
    bb = b_ref[...]
    h = h_s[...]
    c = c_s[...]
    nt = (((1,), (1,)), ((), ()))
    for j in range(TSUB):
        z = (lax.dot_general(wt, emb_ref[j], nt,
                             preferred_element_type=jnp.float32)
             + jnp.dot(ut, h, preferred_element_type=jnp.float32) + bb)
        s_if = jax.nn.sigmoid(z[:2 * H, :])
        i = s_if[:H, :]
        f = s_if[H:, :]
        gg = jnp.tanh(z[2 * H:3 * H, :])
        o = jax.nn.sigmoid(z[3 * H:, :])
        c = f * c + i * gg
        h = o * jnp.tanh(c)
        out_ref[j * H:(j + 1) * H, :] = h
    h_s[...] = h
    c_s[...] = c

    @pl.when(g == NGC - 1)
    def _():
        ht_ref[...] = h
        ct_ref[...] = c


def _tc_lstm_chunk(k, emb_k, ht, ct, wt, ut, b2d, out_buf):
    in_specs = [
        pl.BlockSpec((TSUB, B, D), lambda g: (g, 0, 0)),
        pl.BlockSpec((H, B), lambda g: (0, 0)),
        pl.BlockSpec((H, B), lambda g: (0, 0)),
        pl.BlockSpec((4 * H, D), lambda g: (0, 0)),
        pl.BlockSpec((4 * H, H), lambda g: (0, 0)),
        pl.BlockSpec((4 * H, 1), lambda g: (0, 0)),
    ]
    operands = [emb_k, ht, ct, wt, ut, b2d]
    aliases = {}
    body = _lstm_body
    if out_buf is None:
        def body(emb, h0, c0, w_, u_, bb, out, ho, co, hs, cs):
            return _lstm_body(emb, h0, c0, w_, u_, bb, None, out, ho, co,
                              hs, cs)
    else:
        in_specs.append(pl.BlockSpec(memory_space=pl.ANY))
        operands.append(out_buf)
        aliases = {6: 0}
    return pl.pallas_call(
        body,
        grid=(NGC,),
        in_specs=in_specs,
        out_specs=[
            pl.BlockSpec((TSUB * H, B), lambda g, _k=k: (_k * NGC + g, 0)),
            pl.BlockSpec((H, B), lambda g: (0, 0)),
            pl.BlockSpec((H, B), lambda g: (0, 0)),
        ],
        out_shape=[
            jax.ShapeDtypeStruct((T * H, B), jnp.float32),
            jax.ShapeDtypeStruct((H, B), jnp.float32),
            jax.ShapeDtypeStruct((H, B), jnp.float32),
        ],
        scratch_shapes=[
            pltpu.VMEM((H, B), jnp.float32),
            pltpu.VMEM((H, B), jnp.float32),
        ],
        input_output_aliases=aliases,
    )(*operands)


def kernel(sequence, states_1, states_2, table, W, U, b):
    wt = jnp.transpose(W)
    ut = jnp.transpose(U)
    bt = b.reshape(4 * H, 1)
    ht = jnp.transpose(states_1)
    ct = jnp.transpose(states_2)
    out_buf = None
    for k in range(KCH):
        idx2d = jnp.transpose(
            lax.slice_in_dim(sequence, k * TC, (k + 1) * TC, axis=1)
        ).reshape(32, TC * B // (32 * CH), CH)
        emb_k = _sc_gather_chunk(idx2d, table).reshape(TC, B, D)
        out_buf, ht, ct = _tc_lstm_chunk(k, emb_k, ht, ct, wt, ut, bt,
                                         out_buf)
    out = jnp.transpose(out_buf.reshape(T, H, B), (2, 0, 1))
    return out, jnp.transpose(ht), jnp.transpose(ct)


# speedup vs baseline: 4.2769x; 1.0000x over previous
import functools

import jax
import jax.numpy as jnp
from jax import lax
from jax.experimental import pallas as pl
from jax.experimental.pallas import tpu as pltpu
from jax.experimental.pallas import tpu_sc as plsc

B, T, V, D, H = 1024, 200, 100000, 128, 64
KCH = 4
TC = T // KCH
TSUB = 10
NGC = TC // TSUB
CH = 80
NBUF = 4


def _sc_gather_chunk(idx2d, table):
    info = plsc.get_sparse_core_info()
    nw = info.num_cores * info.num_subcores
    per_w = (TC * B) // nw
    n_ch = per_w // CH
    mesh = plsc.VectorSubcoreMesh(core_axis_name="c", subcore_axis_name="s")

    @functools.partial(
        pl.kernel,
        mesh=mesh,
        out_type=jax.ShapeDtypeStruct((TC * B, D), jnp.float32),
        scratch_types=[
            pltpu.VMEM((n_ch, CH), jnp.int32),
            pltpu.VMEM((NBUF, CH, D), jnp.float32),
        ] + [pltpu.SemaphoreType.DMA] * (2 * NBUF),
    )
    def gather_kernel(idx_hbm, table_hbm, out_hbm, idx_v, bufs, *sems):
        gsem, wsem = sems[:NBUF], sems[NBUF:]
        wid = lax.axis_index("s") * info.num_cores + lax.axis_index("c")
        r0 = wid * per_w
        pltpu.sync_copy(idx_hbm.at[wid], idx_v)

        def g_copy(j):
            r = j % NBUF
            return pltpu.make_async_copy(
                table_hbm.at[idx_v.at[j]], bufs.at[r], gsem[r])

        def w_copy(j):
            r = j % NBUF
            return pltpu.make_async_copy(
                bufs.at[r], out_hbm.at[pl.ds(r0 + j * CH, CH)], wsem[r])

        g_copy(0).start()
        g_copy(1).start()
        for j in range(n_ch):
            g_copy(j).wait()
            w_copy(j).start()
            if j + 2 < n_ch:
                if j >= 2:
                    w_copy(j - 2).wait()
                g_copy(j + 2).start()
        w_copy(n_ch - 4).wait()
        w_copy(n_ch - 3).wait()
        w_copy(n_ch - 2).wait()
        w_copy(n_ch - 1).wait()

    return gather_kernel(idx2d, table)


def _lstm_body(emb_ref, h0_ref, c0_ref, wt_ref, ut_ref, b_ref, obuf_ref,
               out_ref, ht_ref, ct_ref, h_s, c_s):
    del obuf_ref
    g = pl.program_id(0)

    @pl.when(g == 0)
    def _():
        h_s[...] = h0_ref[...]
        c_s[...] = c0_ref[...]

    ut = ut_ref[...]
    wt = wt_ref[...]
    bb = b_ref[...]
    h = h_s[...]
    c = c_s[...]
    nt = (((1,), (1,)), ((), ()))
    for j in range(TSUB):
        z = (lax.dot_general(wt, emb_ref[j], nt,
                             preferred_element_type=jnp.float32)
             + jnp.dot(ut, h, preferred_element_type=jnp.float32) + bb)
        s_if = jax.nn.sigmoid(z[:2 * H, :])
        i = s_if[:H, :]
        f = s_if[H:, :]
        gg = jnp.tanh(z[2 * H:3 * H, :])
        o = jax.nn.sigmoid(z[3 * H:, :])
        c = f * c + i * gg
        h = o * jnp.tanh(c)
        out_ref[j * H:(j + 1) * H, :] = h
    h_s[...] = h
    c_s[...] = c

    @pl.when(g == NGC - 1)
    def _():
        ht_ref[...] = h
        ct_ref[...] = c


def _tc_lstm_chunk(k, emb_k, ht, ct, wt, ut, b2d, out_buf):
    in_specs = [
        pl.BlockSpec((TSUB, B, D), lambda g: (g, 0, 0)),
        pl.BlockSpec((H, B), lambda g: (0, 0)),
        pl.BlockSpec((H, B), lambda g: (0, 0)),
        pl.BlockSpec((4 * H, D), lambda g: (0, 0)),
        pl.BlockSpec((4 * H, H), lambda g: (0, 0)),
        pl.BlockSpec((4 * H, 1), lambda g: (0, 0)),
    ]
    operands = [emb_k, ht, ct, wt, ut, b2d]
    aliases = {}
    body = _lstm_body
    if out_buf is None:
        def body(emb, h0, c0, w_, u_, bb, out, ho, co, hs, cs):
            return _lstm_body(emb, h0, c0, w_, u_, bb, None, out, ho, co,
                              hs, cs)
    else:
        in_specs.append(pl.BlockSpec(memory_space=pl.ANY))
        operands.append(out_buf)
        aliases = {6: 0}
    return pl.pallas_call(
        body,
        grid=(NGC,),
        in_specs=in_specs,
        out_specs=[
            pl.BlockSpec((TSUB * H, B), lambda g, _k=k: (_k * NGC + g, 0)),
            pl.BlockSpec((H, B), lambda g: (0, 0)),
            pl.BlockSpec((H, B), lambda g: (0, 0)),
        ],
        out_shape=[
            jax.ShapeDtypeStruct((T * H, B), jnp.float32),
            jax.ShapeDtypeStruct((H, B), jnp.float32),
            jax.ShapeDtypeStruct((H, B), jnp.float32),
        ],
        scratch_shapes=[
            pltpu.VMEM((H, B), jnp.float32),
            pltpu.VMEM((H, B), jnp.float32),
        ],
        input_output_aliases=aliases,
    )(*operands)


def kernel(sequence, states_1, states_2, table, W, U, b):
    wt = jnp.transpose(W)
    ut = jnp.transpose(U)
    bt = b.reshape(4 * H, 1)
    ht = jnp.transpose(states_1)
    ct = jnp.transpose(states_2)
    out_buf = None
    seq_t = jnp.transpose(sequence)
    for k in range(KCH):
        idx2d = lax.slice_in_dim(seq_t, k * TC, (k + 1) * TC, axis=0
                                 ).reshape(32, TC * B // (32 * CH), CH)
        emb_k = _sc_gather_chunk(idx2d, table).reshape(TC, B, D)
        out_buf, ht, ct = _tc_lstm_chunk(k, emb_k, ht, ct, wt, ut, bt,
                                         out_buf)
    out = jnp.transpose(out_buf.reshape(T, H, B), (2, 0, 1))
    return out, jnp.transpose(ht), jnp.transpose(ct)
